# Initial kernel scaffold; baseline (speedup 1.0000x reference)
import jax
import jax.numpy as jnp
from jax.experimental import pallas as pl


def kernel(z, W):
    raise NotImplementedError("write your pallas kernel here")


# speedup vs baseline: 1.0990x; 1.0990x over previous
import functools

import jax
import jax.numpy as jnp
from jax import lax
from jax.experimental import pallas as pl
from jax.experimental.pallas import tpu as pltpu
from jax.experimental.pallas import tpu_sc as plsc

D_MODEL = 32
CODEBOOK_SIZE = 8192
HALF_K = CODEBOOK_SIZE // 2
COMMITMENT_COST = 0.25

_R = 256


def _dist_argmin_body(zsq_ref, zb_ref, wbt_ref, wsq_ref, idx_ref):
    zb2 = zb_ref[...] * jnp.bfloat16(-2.0)
    mm2 = jnp.dot(zb2, wbt_ref[...], preferred_element_type=jnp.float32)
    t = zsq_ref[...] + wsq_ref[...]
    scores = t + mm2
    s1 = scores[:, :HALF_K]
    s2 = scores[:, HALF_K:]
    m1 = jnp.min(s1, axis=1, keepdims=True)
    m2 = jnp.min(s2, axis=1, keepdims=True)
    iota = lax.broadcasted_iota(jnp.int32, s1.shape, 1)
    i1 = jnp.min(jnp.where(s1 == m1, iota, CODEBOOK_SIZE), axis=1)
    i2 = jnp.min(jnp.where(s2 == m2, iota, CODEBOOK_SIZE), axis=1) + HALF_K
    b1 = m1[:, 0].astype(jnp.bfloat16).astype(jnp.float32)
    idx = jnp.where(m2[:, 0] < b1, i2, i1)
    idx_ref[0, 0, :] = idx


def _argmin_indices(flat, zsq, W, wsq):
    n = flat.shape[0]
    nb = n // _R
    idx3 = pl.pallas_call(
        _dist_argmin_body,
        grid=(nb,),
        in_specs=[
            pl.BlockSpec((_R, 1), lambda i: (i, 0)),
            pl.BlockSpec((_R, D_MODEL), lambda i: (i, 0)),
            pl.BlockSpec((D_MODEL, CODEBOOK_SIZE), lambda i: (0, 0)),
            pl.BlockSpec((1, CODEBOOK_SIZE), lambda i: (0, 0)),
        ],
        out_specs=pl.BlockSpec((1, 1, _R), lambda i: (i, 0, 0)),
        out_shape=jax.ShapeDtypeStruct((nb, 1, _R), jnp.int32),
    )(zsq, flat.astype(jnp.bfloat16), W.T.astype(jnp.bfloat16),
      wsq.reshape(1, CODEBOOK_SIZE))
    return idx3.reshape(n)


def kernel(z, W):
    input_shape = z.shape
    n = z.shape[0] * z.shape[1]
    flat = z.reshape(n, D_MODEL)
    zsq = jnp.sum(flat ** 2, axis=1, keepdims=True)
    wsq = jnp.sum(W ** 2, axis=1)
    indices = _argmin_indices(flat, zsq, W, wsq)
    quantized = jnp.take(W, indices, axis=0)
    diff = quantized - flat
    m = jnp.mean(diff ** 2)
    loss = m + COMMITMENT_COST * m
    quantized_st = (flat + diff).reshape(input_shape)
    return (loss, quantized_st, indices.reshape(input_shape[:-1]))


# speedup vs baseline: 1.2772x; 1.1622x over previous
import functools

import jax
import jax.numpy as jnp
from jax import lax
from jax.experimental import pallas as pl
from jax.experimental.pallas import tpu as pltpu
from jax.experimental.pallas import tpu_sc as plsc

D_MODEL = 32
CODEBOOK_SIZE = 8192
HALF_K = CODEBOOK_SIZE // 2
COMMITMENT_COST = 0.25

_R = 256


def _dist_argmin_body(zsq_ref, zb_ref, wbt_ref, wsq_ref, idx_ref):
    zb2 = zb_ref[...] * jnp.bfloat16(-2.0)
    mm2 = jnp.dot(zb2, wbt_ref[...], preferred_element_type=jnp.float32)
    t = zsq_ref[...] + wsq_ref[...]
    scores = t + mm2
    s1 = scores[:, :HALF_K]
    s2 = scores[:, HALF_K:]
    m1 = jnp.min(s1, axis=1, keepdims=True)
    m2 = jnp.min(s2, axis=1, keepdims=True)
    iota = lax.broadcasted_iota(jnp.int32, s1.shape, 1)
    i1 = jnp.min(jnp.where(s1 == m1, iota, CODEBOOK_SIZE), axis=1)
    i2 = jnp.min(jnp.where(s2 == m2, iota, CODEBOOK_SIZE), axis=1) + HALF_K
    b1 = m1[:, 0].astype(jnp.bfloat16).astype(jnp.float32)
    idx = jnp.where(m2[:, 0] < b1, i2, i1)
    idx_ref[0, 0, :] = idx


def _argmin_indices(flat, zsq, W, wsq):
    n = flat.shape[0]
    nb = n // _R
    idx3 = pl.pallas_call(
        _dist_argmin_body,
        grid=(nb,),
        in_specs=[
            pl.BlockSpec((_R, 1), lambda i: (i, 0)),
            pl.BlockSpec((_R, D_MODEL), lambda i: (i, 0)),
            pl.BlockSpec((D_MODEL, CODEBOOK_SIZE), lambda i: (0, 0)),
            pl.BlockSpec((1, CODEBOOK_SIZE), lambda i: (0, 0)),
        ],
        out_specs=pl.BlockSpec((1, 1, _R), lambda i: (i, 0, 0)),
        out_shape=jax.ShapeDtypeStruct((nb, 1, _R), jnp.int32),
    )(zsq, flat.astype(jnp.bfloat16), W.T.astype(jnp.bfloat16),
      wsq.reshape(1, CODEBOOK_SIZE))
    return idx3.reshape(n)


_NC = 2
_NS = 16
_NW = _NC * _NS
_GR = 128
_CH = 512


def _sc_gather_body(w_hbm, z_hbm, idx_hbm, qst_hbm, part_hbm,
                    idx_v, rows_v, z_v, acc_v, sem):
    wid = lax.axis_index("s") * _NC + lax.axis_index("c")
    n_rows = z_hbm.shape[0]
    rpw = n_rows // _NW
    nch = rpw // _CH
    ng = _CH // _GR
    acc = jnp.zeros((16,), jnp.float32)
    pltpu.sync_copy(idx_hbm.at[pl.ds(wid * (rpw // _GR), rpw // _GR)], idx_v)
    for c in range(nch):
        row0 = wid * rpw + c * _CH
        copies = [
            pltpu.async_copy(w_hbm.at[idx_v.at[c * ng + g]],
                             rows_v.at[pl.ds(g * _GR, _GR)], sem)
            for g in range(ng)
        ]
        pltpu.sync_copy(z_hbm.at[pl.ds(row0, _CH)], z_v)
        for cp in copies:
            cp.wait()

        def body(i, acc):
            for j in range(2):
                q = rows_v[i, pl.ds(j * 16, 16)]
                f = z_v[i, pl.ds(j * 16, 16)]
                dqf = q - f
                acc = acc + dqf * dqf
                rows_v[i, pl.ds(j * 16, 16)] = f + dqf
            return acc

        acc = lax.fori_loop(0, _CH, body, acc)
        pltpu.sync_copy(rows_v, qst_hbm.at[pl.ds(row0, _CH)])
    acc_v[...] = acc
    pltpu.sync_copy(acc_v, part_hbm.at[wid])


def _sc_gather_loss(W, flat, indices):
    n = flat.shape[0]
    mesh = plsc.VectorSubcoreMesh(core_axis_name="c", subcore_axis_name="s")
    run = functools.partial(
        pl.kernel,
        out_type=(jax.ShapeDtypeStruct((n, D_MODEL), jnp.float32),
                  jax.ShapeDtypeStruct((_NW, 16), jnp.float32)),
        mesh=mesh,
        scratch_types=[
            pltpu.VMEM((n // _NW // _GR, _GR), jnp.int32),
            pltpu.VMEM((_CH, D_MODEL), jnp.float32),
            pltpu.VMEM((_CH, D_MODEL), jnp.float32),
            pltpu.VMEM((16,), jnp.float32),
            pltpu.SemaphoreType.DMA,
        ],
        compiler_params=pltpu.CompilerParams(use_tc_tiling_on_sc=False),
    )(_sc_gather_body)
    return run(W, flat, indices.reshape(n // _GR, _GR))


def kernel(z, W):
    input_shape = z.shape
    n = z.shape[0] * z.shape[1]
    flat = z.reshape(n, D_MODEL)
    zsq = jnp.sum(flat ** 2, axis=1, keepdims=True)
    wsq = jnp.sum(W ** 2, axis=1)
    indices = _argmin_indices(flat, zsq, W, wsq)
    quantized_st, part = _sc_gather_loss(W, flat, indices)
    m = jnp.sum(part) / (n * D_MODEL)
    loss = m + COMMITMENT_COST * m
    return (loss, quantized_st.reshape(input_shape),
            indices.reshape(input_shape[:-1]))


# speedup vs baseline: 1.4333x; 1.1222x over previous
import functools

import jax
import jax.numpy as jnp
from jax import lax
from jax.experimental import pallas as pl
from jax.experimental.pallas import tpu as pltpu
from jax.experimental.pallas import tpu_sc as plsc

D_MODEL = 32
CODEBOOK_SIZE = 8192
HALF_K = CODEBOOK_SIZE // 2
COMMITMENT_COST = 0.25

_R = 512


def _dist_argmin_body(zsq_ref, zb_ref, wbt_ref, wsq_ref, iota_ref, idx_ref):
    zb2 = zb_ref[...] * jnp.bfloat16(-2.0)
    mm2 = jnp.dot(zb2, wbt_ref[...], preferred_element_type=jnp.float32)
    t = zsq_ref[...] + wsq_ref[...]
    scores = t + mm2
    s1 = scores[:, :HALF_K]
    s2 = scores[:, HALF_K:]
    m1 = jnp.min(s1, axis=1, keepdims=True)
    m2 = jnp.min(s2, axis=1, keepdims=True)
    iota = iota_ref[...]
    big = jnp.float32(CODEBOOK_SIZE)
    i1 = jnp.min(jnp.where(s1 == m1, iota[:, :HALF_K], big), axis=1)
    i2 = jnp.min(jnp.where(s2 == m2, iota[:, HALF_K:], big), axis=1)
    b1 = m1[:, 0].astype(jnp.bfloat16).astype(jnp.float32)
    idx = jnp.where(m2[:, 0] < b1, i2, i1).astype(jnp.int32)
    idx_ref[0, 0, :] = idx


def _argmin_indices(flat, zsq, W, wsq):
    n = flat.shape[0]
    nb = n // _R
    idx3 = pl.pallas_call(
        _dist_argmin_body,
        grid=(nb,),
        in_specs=[
            pl.BlockSpec((_R, 1), lambda i: (i, 0)),
            pl.BlockSpec((_R, D_MODEL), lambda i: (i, 0)),
            pl.BlockSpec((D_MODEL, CODEBOOK_SIZE), lambda i: (0, 0)),
            pl.BlockSpec((1, CODEBOOK_SIZE), lambda i: (0, 0)),
            pl.BlockSpec((1, CODEBOOK_SIZE), lambda i: (0, 0)),
        ],
        out_specs=pl.BlockSpec((1, 1, _R), lambda i: (i, 0, 0)),
        out_shape=jax.ShapeDtypeStruct((nb, 1, _R), jnp.int32),
    )(zsq, flat.astype(jnp.bfloat16), W.T.astype(jnp.bfloat16),
      wsq.reshape(1, CODEBOOK_SIZE),
      jnp.arange(CODEBOOK_SIZE, dtype=jnp.float32).reshape(1, CODEBOOK_SIZE))
    return idx3.reshape(n)


_NC = 2
_NS = 16
_NW = _NC * _NS
_GR = 128
_CH = 512


def _sc_gather_body(w_hbm, z_hbm, idx_hbm, qst_hbm, part_hbm,
                    idx_v, rows_v, z_v, acc_v, sem):
    wid = lax.axis_index("s") * _NC + lax.axis_index("c")
    n_rows = z_hbm.shape[0]
    rpw = n_rows // _NW
    nch = rpw // _CH
    ng = _CH // _GR
    acc = jnp.zeros((16,), jnp.float32)
    pltpu.sync_copy(idx_hbm.at[pl.ds(wid * (rpw // _GR), rpw // _GR)], idx_v)
    for c in range(nch):
        row0 = wid * rpw + c * _CH
        copies = [
            pltpu.async_copy(w_hbm.at[idx_v.at[c * ng + g]],
                             rows_v.at[pl.ds(g * _GR, _GR)], sem)
            for g in range(ng)
        ]
        pltpu.sync_copy(z_hbm.at[pl.ds(row0, _CH)], z_v)
        for cp in copies:
            cp.wait()

        def body(i, acc):
            for j in range(2):
                q = rows_v[i, pl.ds(j * 16, 16)]
                f = z_v[i, pl.ds(j * 16, 16)]
                dqf = q - f
                acc = acc + dqf * dqf
                rows_v[i, pl.ds(j * 16, 16)] = f + dqf
            return acc

        acc = lax.fori_loop(0, _CH, body, acc)
        pltpu.sync_copy(rows_v, qst_hbm.at[pl.ds(row0, _CH)])
    acc_v[...] = acc
    pltpu.sync_copy(acc_v, part_hbm.at[wid])


def _sc_gather_loss(W, flat, indices):
    n = flat.shape[0]
    mesh = plsc.VectorSubcoreMesh(core_axis_name="c", subcore_axis_name="s")
    run = functools.partial(
        pl.kernel,
        out_type=(jax.ShapeDtypeStruct((n, D_MODEL), jnp.float32),
                  jax.ShapeDtypeStruct((_NW, 16), jnp.float32)),
        mesh=mesh,
        scratch_types=[
            pltpu.VMEM((n // _NW // _GR, _GR), jnp.int32),
            pltpu.VMEM((_CH, D_MODEL), jnp.float32),
            pltpu.VMEM((_CH, D_MODEL), jnp.float32),
            pltpu.VMEM((16,), jnp.float32),
            pltpu.SemaphoreType.DMA,
        ],
        compiler_params=pltpu.CompilerParams(use_tc_tiling_on_sc=False),
    )(_sc_gather_body)
    return run(W, flat, indices.reshape(n // _GR, _GR))


def kernel(z, W):
    input_shape = z.shape
    n = z.shape[0] * z.shape[1]
    flat = z.reshape(n, D_MODEL)
    zsq = jnp.sum(flat ** 2, axis=1, keepdims=True)
    wsq = jnp.sum(W ** 2, axis=1)
    indices = _argmin_indices(flat, zsq, W, wsq)
    quantized_st, part = _sc_gather_loss(W, flat, indices)
    m = jnp.sum(part) / (n * D_MODEL)
    loss = m + COMMITMENT_COST * m
    return (loss, quantized_st.reshape(input_shape),
            indices.reshape(input_shape[:-1]))


# speedup vs baseline: 1.4471x; 1.0096x over previous
import functools

import jax
import jax.numpy as jnp
from jax import lax
from jax.experimental import pallas as pl
from jax.experimental.pallas import tpu as pltpu
from jax.experimental.pallas import tpu_sc as plsc

D_MODEL = 32
CODEBOOK_SIZE = 8192
HALF_K = CODEBOOK_SIZE // 2
COMMITMENT_COST = 0.25

_R = 1024


def _dist_argmin_body(zsq_ref, zb_ref, wbt_ref, wsq_ref, iota_ref, idx_ref):
    zb2 = zb_ref[...] * jnp.bfloat16(-2.0)
    mm2 = jnp.dot(zb2, wbt_ref[...], preferred_element_type=jnp.float32)
    t = zsq_ref[...] + wsq_ref[...]
    scores = t + mm2
    s1 = scores[:, :HALF_K]
    s2 = scores[:, HALF_K:]
    m1 = jnp.min(s1, axis=1, keepdims=True)
    m2 = jnp.min(s2, axis=1, keepdims=True)
    iota = iota_ref[...]
    big = jnp.float32(CODEBOOK_SIZE)
    i1 = jnp.min(jnp.where(s1 == m1, iota[:, :HALF_K], big), axis=1)
    i2 = jnp.min(jnp.where(s2 == m2, iota[:, HALF_K:], big), axis=1)
    b1 = m1[:, 0].astype(jnp.bfloat16).astype(jnp.float32)
    idx = jnp.where(m2[:, 0] < b1, i2, i1).astype(jnp.int32)
    idx_ref[0, 0, :] = idx


def _argmin_indices(flat, zsq, W, wsq):
    n = flat.shape[0]
    nb = n // _R
    idx3 = pl.pallas_call(
        _dist_argmin_body,
        grid=(nb,),
        in_specs=[
            pl.BlockSpec((_R, 1), lambda i: (i, 0)),
            pl.BlockSpec((_R, D_MODEL), lambda i: (i, 0)),
            pl.BlockSpec((D_MODEL, CODEBOOK_SIZE), lambda i: (0, 0)),
            pl.BlockSpec((1, CODEBOOK_SIZE), lambda i: (0, 0)),
            pl.BlockSpec((1, CODEBOOK_SIZE), lambda i: (0, 0)),
        ],
        out_specs=pl.BlockSpec((1, 1, _R), lambda i: (i, 0, 0)),
        out_shape=jax.ShapeDtypeStruct((nb, 1, _R), jnp.int32),
    )(zsq, flat.astype(jnp.bfloat16), W.T.astype(jnp.bfloat16),
      wsq.reshape(1, CODEBOOK_SIZE),
      jnp.arange(CODEBOOK_SIZE, dtype=jnp.float32).reshape(1, CODEBOOK_SIZE))
    return idx3.reshape(n)


_NC = 2
_NS = 16
_NW = _NC * _NS
_GR = 128
_CH = 512


def _sc_gather_body(w_hbm, z_hbm, idx_hbm, qst_hbm, part_hbm,
                    idx_v, rows_v, z_v, acc_v, sem):
    wid = lax.axis_index("s") * _NC + lax.axis_index("c")
    n_rows = z_hbm.shape[0]
    rpw = n_rows // _NW
    nch = rpw // _CH
    ng = _CH // _GR
    acc = jnp.zeros((16,), jnp.float32)
    pltpu.sync_copy(idx_hbm.at[pl.ds(wid * (rpw // _GR), rpw // _GR)], idx_v)
    for c in range(nch):
        row0 = wid * rpw + c * _CH
        copies = [
            pltpu.async_copy(w_hbm.at[idx_v.at[c * ng + g]],
                             rows_v.at[pl.ds(g * _GR, _GR)], sem)
            for g in range(ng)
        ]
        pltpu.sync_copy(z_hbm.at[pl.ds(row0, _CH)], z_v)
        for cp in copies:
            cp.wait()

        def body(i, acc):
            for j in range(2):
                q = rows_v[i, pl.ds(j * 16, 16)]
                f = z_v[i, pl.ds(j * 16, 16)]
                dqf = q - f
                acc = acc + dqf * dqf
                rows_v[i, pl.ds(j * 16, 16)] = f + dqf
            return acc

        acc = lax.fori_loop(0, _CH, body, acc)
        pltpu.sync_copy(rows_v, qst_hbm.at[pl.ds(row0, _CH)])
    acc_v[...] = acc
    pltpu.sync_copy(acc_v, part_hbm.at[wid])


def _sc_gather_loss(W, flat, indices):
    n = flat.shape[0]
    mesh = plsc.VectorSubcoreMesh(core_axis_name="c", subcore_axis_name="s")
    run = functools.partial(
        pl.kernel,
        out_type=(jax.ShapeDtypeStruct((n, D_MODEL), jnp.float32),
                  jax.ShapeDtypeStruct((_NW, 16), jnp.float32)),
        mesh=mesh,
        scratch_types=[
            pltpu.VMEM((n // _NW // _GR, _GR), jnp.int32),
            pltpu.VMEM((_CH, D_MODEL), jnp.float32),
            pltpu.VMEM((_CH, D_MODEL), jnp.float32),
            pltpu.VMEM((16,), jnp.float32),
            pltpu.SemaphoreType.DMA,
        ],
        compiler_params=pltpu.CompilerParams(use_tc_tiling_on_sc=False),
    )(_sc_gather_body)
    return run(W, flat, indices.reshape(n // _GR, _GR))


def kernel(z, W):
    input_shape = z.shape
    n = z.shape[0] * z.shape[1]
    flat = z.reshape(n, D_MODEL)
    zsq = jnp.sum(flat ** 2, axis=1, keepdims=True)
    wsq = jnp.sum(W ** 2, axis=1)
    indices = _argmin_indices(flat, zsq, W, wsq)
    quantized_st, part = _sc_gather_loss(W, flat, indices)
    m = jnp.sum(part) / (n * D_MODEL)
    loss = m + COMMITMENT_COST * m
    return (loss, quantized_st.reshape(input_shape),
            indices.reshape(input_shape[:-1]))


# speedup vs baseline: 1.5095x; 1.0431x over previous
import functools

import jax
import jax.numpy as jnp
from jax import lax
from jax.experimental import pallas as pl
from jax.experimental.pallas import tpu as pltpu
from jax.experimental.pallas import tpu_sc as plsc

D_MODEL = 32
CODEBOOK_SIZE = 8192
HALF_K = CODEBOOK_SIZE // 2
COMMITMENT_COST = 0.25

_R = 1024


def _dist_argmin_body(z_ref, wbt_ref, wsq_ref, iota_ref, idx_ref):
    zf = z_ref[...]
    zb2 = zf.astype(jnp.bfloat16) * jnp.bfloat16(-2.0)
    mm2 = jnp.dot(zb2, wbt_ref[...], preferred_element_type=jnp.float32)
    zsq = jnp.sum(zf ** 2, axis=1, keepdims=True)
    t = zsq + wsq_ref[...]
    scores = t + mm2
    s1 = scores[:, :HALF_K]
    s2 = scores[:, HALF_K:]
    m1 = jnp.min(s1, axis=1, keepdims=True)
    m2 = jnp.min(s2, axis=1, keepdims=True)
    iota = iota_ref[...]
    big = jnp.float32(CODEBOOK_SIZE)
    i1 = jnp.min(jnp.where(s1 == m1, iota[:, :HALF_K], big), axis=1)
    i2 = jnp.min(jnp.where(s2 == m2, iota[:, HALF_K:], big), axis=1)
    b1 = m1[:, 0].astype(jnp.bfloat16).astype(jnp.float32)
    idx = jnp.where(m2[:, 0] < b1, i2, i1).astype(jnp.int32)
    idx_ref[0, 0, :] = idx


def _argmin_indices(flat, W, wsq):
    n = flat.shape[0]
    nb = n // _R
    idx3 = pl.pallas_call(
        _dist_argmin_body,
        grid=(nb,),
        in_specs=[
            pl.BlockSpec((_R, D_MODEL), lambda i: (i, 0)),
            pl.BlockSpec((D_MODEL, CODEBOOK_SIZE), lambda i: (0, 0)),
            pl.BlockSpec((1, CODEBOOK_SIZE), lambda i: (0, 0)),
            pl.BlockSpec((1, CODEBOOK_SIZE), lambda i: (0, 0)),
        ],
        out_specs=pl.BlockSpec((1, 1, _R), lambda i: (i, 0, 0)),
        out_shape=jax.ShapeDtypeStruct((nb, 1, _R), jnp.int32),
    )(flat, W.T.astype(jnp.bfloat16),
      wsq.reshape(1, CODEBOOK_SIZE),
      jnp.arange(CODEBOOK_SIZE, dtype=jnp.float32).reshape(1, CODEBOOK_SIZE))
    return idx3.reshape(n)


_NC = 2
_NS = 16
_NW = _NC * _NS
_GR = 128
_CH = 512


def _sc_gather_body(w_hbm, z_hbm, idx_hbm, qst_hbm, part_hbm,
                    idx_v, rows_v, z_v, acc_v, sem):
    wid = lax.axis_index("s") * _NC + lax.axis_index("c")
    n_rows = z_hbm.shape[0]
    rpw = n_rows // _NW
    nch = rpw // _CH
    ng = _CH // _GR
    acc = jnp.zeros((16,), jnp.float32)
    pltpu.sync_copy(idx_hbm.at[pl.ds(wid * (rpw // _GR), rpw // _GR)], idx_v)
    for c in range(nch):
        row0 = wid * rpw + c * _CH
        copies = [
            pltpu.async_copy(w_hbm.at[idx_v.at[c * ng + g]],
                             rows_v.at[pl.ds(g * _GR, _GR)], sem)
            for g in range(ng)
        ]
        pltpu.sync_copy(z_hbm.at[pl.ds(row0, _CH)], z_v)
        for cp in copies:
            cp.wait()

        def body(i, acc):
            for j in range(2):
                q = rows_v[i, pl.ds(j * 16, 16)]
                f = z_v[i, pl.ds(j * 16, 16)]
                dqf = q - f
                acc = acc + dqf * dqf
                rows_v[i, pl.ds(j * 16, 16)] = f + dqf
            return acc

        acc = lax.fori_loop(0, _CH, body, acc)
        pltpu.sync_copy(rows_v, qst_hbm.at[pl.ds(row0, _CH)])
    acc_v[...] = acc
    pltpu.sync_copy(acc_v, part_hbm.at[wid])


def _sc_gather_loss(W, flat, indices):
    n = flat.shape[0]
    mesh = plsc.VectorSubcoreMesh(core_axis_name="c", subcore_axis_name="s")
    run = functools.partial(
        pl.kernel,
        out_type=(jax.ShapeDtypeStruct((n, D_MODEL), jnp.float32),
                  jax.ShapeDtypeStruct((_NW, 16), jnp.float32)),
        mesh=mesh,
        scratch_types=[
            pltpu.VMEM((n // _NW // _GR, _GR), jnp.int32),
            pltpu.VMEM((_CH, D_MODEL), jnp.float32),
            pltpu.VMEM((_CH, D_MODEL), jnp.float32),
            pltpu.VMEM((16,), jnp.float32),
            pltpu.SemaphoreType.DMA,
        ],
        compiler_params=pltpu.CompilerParams(use_tc_tiling_on_sc=False),
    )(_sc_gather_body)
    return run(W, flat, indices.reshape(n // _GR, _GR))


def kernel(z, W):
    input_shape = z.shape
    n = z.shape[0] * z.shape[1]
    flat = z.reshape(n, D_MODEL)
    wsq = jnp.sum(W ** 2, axis=1)
    indices = _argmin_indices(flat, W, wsq)
    quantized_st, part = _sc_gather_loss(W, flat, indices)
    m = jnp.sum(part) / (n * D_MODEL)
    loss = m + COMMITMENT_COST * m
    return (loss, quantized_st.reshape(input_shape),
            indices.reshape(input_shape[:-1]))


# speedup vs baseline: 1.5106x; 1.0007x over previous
import functools

import jax
import jax.numpy as jnp
from jax import lax
from jax.experimental import pallas as pl
from jax.experimental.pallas import tpu as pltpu
from jax.experimental.pallas import tpu_sc as plsc

D_MODEL = 32
CODEBOOK_SIZE = 8192
HALF_K = CODEBOOK_SIZE // 2
COMMITMENT_COST = 0.25

_R = 2048


def _dist_argmin_body(z_ref, wbt_ref, wsq_ref, iota_ref, idx_ref):
    zf = z_ref[...]
    zb2 = zf.astype(jnp.bfloat16) * jnp.bfloat16(-2.0)
    mm2 = jnp.dot(zb2, wbt_ref[...], preferred_element_type=jnp.float32)
    zsq = jnp.sum(zf ** 2, axis=1, keepdims=True)
    t = zsq + wsq_ref[...]
    scores = t + mm2
    s1 = scores[:, :HALF_K]
    s2 = scores[:, HALF_K:]
    m1 = jnp.min(s1, axis=1, keepdims=True)
    m2 = jnp.min(s2, axis=1, keepdims=True)
    iota = iota_ref[...]
    big = jnp.float32(CODEBOOK_SIZE)
    i1 = jnp.min(jnp.where(s1 == m1, iota[:, :HALF_K], big), axis=1)
    i2 = jnp.min(jnp.where(s2 == m2, iota[:, HALF_K:], big), axis=1)
    b1 = m1[:, 0].astype(jnp.bfloat16).astype(jnp.float32)
    idx = jnp.where(m2[:, 0] < b1, i2, i1).astype(jnp.int32)
    idx_ref[0, 0, :] = idx


def _argmin_indices(flat, W, wsq):
    n = flat.shape[0]
    nb = n // _R
    idx3 = pl.pallas_call(
        _dist_argmin_body,
        grid=(nb,),
        in_specs=[
            pl.BlockSpec((_R, D_MODEL), lambda i: (i, 0)),
            pl.BlockSpec((D_MODEL, CODEBOOK_SIZE), lambda i: (0, 0)),
            pl.BlockSpec((1, CODEBOOK_SIZE), lambda i: (0, 0)),
            pl.BlockSpec((1, CODEBOOK_SIZE), lambda i: (0, 0)),
        ],
        out_specs=pl.BlockSpec((1, 1, _R), lambda i: (i, 0, 0)),
        out_shape=jax.ShapeDtypeStruct((nb, 1, _R), jnp.int32),
    )(flat, W.T.astype(jnp.bfloat16),
      wsq.reshape(1, CODEBOOK_SIZE),
      jnp.arange(CODEBOOK_SIZE, dtype=jnp.float32).reshape(1, CODEBOOK_SIZE))
    return idx3.reshape(n)


_NC = 2
_NS = 16
_NW = _NC * _NS
_GR = 128
_CH = 512


def _sc_gather_body(w_hbm, z_hbm, idx_hbm, qst_hbm, part_hbm,
                    idx_v, rows_v, z_v, acc_v, sem):
    wid = lax.axis_index("s") * _NC + lax.axis_index("c")
    n_rows = z_hbm.shape[0]
    rpw = n_rows // _NW
    nch = rpw // _CH
    ng = _CH // _GR
    acc = jnp.zeros((16,), jnp.float32)
    pltpu.sync_copy(idx_hbm.at[pl.ds(wid * (rpw // _GR), rpw // _GR)], idx_v)
    for c in range(nch):
        row0 = wid * rpw + c * _CH
        copies = [
            pltpu.async_copy(w_hbm.at[idx_v.at[c * ng + g]],
                             rows_v.at[pl.ds(g * _GR, _GR)], sem)
            for g in range(ng)
        ]
        pltpu.sync_copy(z_hbm.at[pl.ds(row0, _CH)], z_v)
        for cp in copies:
            cp.wait()

        def body(i, acc):
            for j in range(2):
                q = rows_v[i, pl.ds(j * 16, 16)]
                f = z_v[i, pl.ds(j * 16, 16)]
                dqf = q - f
                acc = acc + dqf * dqf
                rows_v[i, pl.ds(j * 16, 16)] = f + dqf
            return acc

        acc = lax.fori_loop(0, _CH, body, acc)
        pltpu.sync_copy(rows_v, qst_hbm.at[pl.ds(row0, _CH)])
    acc_v[...] = acc
    pltpu.sync_copy(acc_v, part_hbm.at[wid])


def _sc_gather_loss(W, flat, indices):
    n = flat.shape[0]
    mesh = plsc.VectorSubcoreMesh(core_axis_name="c", subcore_axis_name="s")
    run = functools.partial(
        pl.kernel,
        out_type=(jax.ShapeDtypeStruct((n, D_MODEL), jnp.float32),
                  jax.ShapeDtypeStruct((_NW, 16), jnp.float32)),
        mesh=mesh,
        scratch_types=[
            pltpu.VMEM((n // _NW // _GR, _GR), jnp.int32),
            pltpu.VMEM((_CH, D_MODEL), jnp.float32),
            pltpu.VMEM((_CH, D_MODEL), jnp.float32),
            pltpu.VMEM((16,), jnp.float32),
            pltpu.SemaphoreType.DMA,
        ],
        compiler_params=pltpu.CompilerParams(use_tc_tiling_on_sc=False),
    )(_sc_gather_body)
    return run(W, flat, indices.reshape(n // _GR, _GR))


def kernel(z, W):
    input_shape = z.shape
    n = z.shape[0] * z.shape[1]
    flat = z.reshape(n, D_MODEL)
    wsq = jnp.sum(W ** 2, axis=1)
    indices = _argmin_indices(flat, W, wsq)
    quantized_st, part = _sc_gather_loss(W, flat, indices)
    m = jnp.sum(part) / (n * D_MODEL)
    loss = m + COMMITMENT_COST * m
    return (loss, quantized_st.reshape(input_shape),
            indices.reshape(input_shape[:-1]))
